# Initial kernel scaffold; baseline (speedup 1.0000x reference)
import functools

import jax
import jax.numpy as jnp
from jax import lax
from jax.experimental import pallas as pl
from jax.experimental.pallas import tpu as pltpu
from jax.experimental.pallas import tpu_sc as plsc

N = 10000
E = 320000
D_IN = 128
D_HID = 128
D_OUT = 64
KSTEPS = 10
ALPHA = 0.1

NC = 2
NS = 16
NW = NC * NS
EPW = E // NW
CHUNK = 80
NCHUNK = EPW // CHUNK
RPT = N // NS
ZROWS = 25

BR = 400
GRID = N // BR


def _mesh():
    return plsc.VectorSubcoreMesh(core_axis_name="c", subcore_axis_name="s")


def _deg_body(col3, degp, cbuf, dloc, *, ones16, zeros16):
    c = lax.axis_index("c")
    s = lax.axis_index("s")
    wid = c * NS + s

    def zero_body(r, carry):
        dloc[pl.ds(r * 16, 16)] = zeros16
        return carry

    lax.fori_loop(0, N // 16, zero_body, 0)
    pltpu.sync_copy(col3.at[wid], cbuf)

    def body(i, carry):
        for j in range(CHUNK // 16):
            idx16 = cbuf[i, pl.ds(j * 16, 16)]
            plsc.addupdate_scatter(dloc, [idx16], ones16)
        return carry

    lax.fori_loop(0, NCHUNK, body, 0)
    pltpu.sync_copy(dloc, degp.at[wid])


def _deg_sc(col3):
    ones16 = jnp.full((16,), 1.0, jnp.float32)
    zeros16 = jnp.zeros((16,), jnp.float32)
    k = pl.kernel(
        functools.partial(_deg_body, ones16=ones16, zeros16=zeros16),
        out_type=jax.ShapeDtypeStruct((NW, N), jnp.float32),
        mesh=_mesh(),
        scratch_types=[
            pltpu.VMEM((NCHUNK, CHUNK), jnp.int32),
            pltpu.VMEM((N,), jnp.float32),
        ],
    )
    return k(col3)


def _scatter_body(row3, col3, ht_h, pacc, row_v, col_v, rows_v, zer_v,
                  acc_sh, sem):
    c = lax.axis_index("c")
    s = lax.axis_index("s")
    wid = c * NS + s

    for r in range(ZROWS):
        for j in range(D_HID // 16):
            zer_v[r, pl.ds(j * 16, 16)] = jnp.zeros((16,), jnp.float32)
    for j in range(RPT // ZROWS):
        pltpu.sync_copy(zer_v, acc_sh.at[pl.ds(s * RPT + j * ZROWS, ZROWS)])
    pltpu.sync_copy(row3.at[wid], row_v)
    pltpu.sync_copy(col3.at[wid], col_v)
    plsc.subcore_barrier()

    def body(i, carry):
        pltpu.async_copy(ht_h.at[row_v.at[i]], rows_v, sem).wait()
        pltpu.sync_copy(rows_v, acc_sh.at[col_v.at[i]], add=True)
        return carry

    lax.fori_loop(0, NCHUNK, body, 0)
    plsc.subcore_barrier()
    pltpu.sync_copy(acc_sh.at[pl.ds(s * RPT, RPT)],
                    pacc.at[c, pl.ds(s * RPT, RPT)])


def _scatter_sc(row3, col3, ht):
    k = pl.kernel(
        _scatter_body,
        out_type=jax.ShapeDtypeStruct((NC, N, D_HID), jnp.float32),
        mesh=_mesh(),
        scratch_types=[
            pltpu.VMEM((NCHUNK, CHUNK), jnp.int32),
            pltpu.VMEM((NCHUNK, CHUNK), jnp.int32),
            pltpu.VMEM((CHUNK, D_HID), jnp.float32),
            pltpu.VMEM((ZROWS, D_HID), jnp.float32),
            pltpu.VMEM_SHARED((N, D_HID), jnp.float32),
            pltpu.SemaphoreType.DMA,
        ],
    )
    return k(row3, col3, ht)


def _mlp_body(x_ref, w1_ref, b1_ref, w2_ref, b2_ref, h_ref):
    h = jnp.maximum(
        jnp.dot(x_ref[...], w1_ref[...], preferred_element_type=jnp.float32)
        + b1_ref[...], 0.0)
    h_ref[...] = (
        jnp.dot(h, w2_ref[...], preferred_element_type=jnp.float32)
        + b2_ref[...])


def _mlp(x, W1, b1, W2, b2):
    return pl.pallas_call(
        _mlp_body,
        grid=(GRID,),
        in_specs=[
            pl.BlockSpec((BR, D_IN), lambda i: (i, 0)),
            pl.BlockSpec((D_IN, D_HID), lambda i: (0, 0)),
            pl.BlockSpec((1, D_HID), lambda i: (0, 0)),
            pl.BlockSpec((D_HID, D_HID), lambda i: (0, 0)),
            pl.BlockSpec((1, D_HID), lambda i: (0, 0)),
        ],
        out_specs=pl.BlockSpec((BR, D_HID), lambda i: (i, 0)),
        out_shape=jax.ShapeDtypeStruct((N, D_HID), jnp.float32),
    )(x, W1, b1, W2, b2)


def _prep_body(degp_ref, h_ref, dis_ref, ht_ref):
    ones = jnp.ones((NW, 1), jnp.float32)
    deg = lax.dot_general(degp_ref[...], ones, (((0,), (0,)), ((), ())),
                          preferred_element_type=jnp.float32)
    dis = lax.rsqrt(deg + 1.0)
    dis_ref[...] = dis
    ht_ref[...] = dis * h_ref[...]


def _prep(degp, h):
    return pl.pallas_call(
        _prep_body,
        grid=(GRID,),
        in_specs=[
            pl.BlockSpec((NW, BR), lambda i: (0, i)),
            pl.BlockSpec((BR, D_HID), lambda i: (i, 0)),
        ],
        out_specs=[
            pl.BlockSpec((BR, 1), lambda i: (i, 0)),
            pl.BlockSpec((BR, D_HID), lambda i: (i, 0)),
        ],
        out_shape=[
            jax.ShapeDtypeStruct((N, 1), jnp.float32),
            jax.ShapeDtypeStruct((N, D_HID), jnp.float32),
        ],
    )(degp, h)


def _combine_body(pacc_ref, ht_ref, h0_ref, dis_ref, out_ref):
    acc = pacc_ref[0] + pacc_ref[1]
    hk = ((1.0 - ALPHA) * dis_ref[...] * (acc + ht_ref[...])
          + ALPHA * h0_ref[...])
    out_ref[...] = dis_ref[...] * hk


def _combine(pacc, ht, h0, dis):
    return pl.pallas_call(
        _combine_body,
        grid=(GRID,),
        in_specs=[
            pl.BlockSpec((NC, BR, D_HID), lambda i: (0, i, 0)),
            pl.BlockSpec((BR, D_HID), lambda i: (i, 0)),
            pl.BlockSpec((BR, D_HID), lambda i: (i, 0)),
            pl.BlockSpec((BR, 1), lambda i: (i, 0)),
        ],
        out_specs=pl.BlockSpec((BR, D_HID), lambda i: (i, 0)),
        out_shape=jax.ShapeDtypeStruct((N, D_HID), jnp.float32),
    )(pacc, ht, h0, dis)


def _final_body(pacc_ref, ht_ref, h0_ref, dis_ref, w3_ref, b3_ref, out_ref):
    acc = pacc_ref[0] + pacc_ref[1]
    hk = ((1.0 - ALPHA) * dis_ref[...] * (acc + ht_ref[...])
          + ALPHA * h0_ref[...])
    out_ref[...] = (
        jnp.dot(hk, w3_ref[...], preferred_element_type=jnp.float32)
        + b3_ref[...])


def _final(pacc, ht, h0, dis, W3, b3):
    return pl.pallas_call(
        _final_body,
        grid=(GRID,),
        in_specs=[
            pl.BlockSpec((NC, BR, D_HID), lambda i: (0, i, 0)),
            pl.BlockSpec((BR, D_HID), lambda i: (i, 0)),
            pl.BlockSpec((BR, D_HID), lambda i: (i, 0)),
            pl.BlockSpec((BR, 1), lambda i: (i, 0)),
            pl.BlockSpec((D_HID, D_OUT), lambda i: (0, 0)),
            pl.BlockSpec((1, D_OUT), lambda i: (0, 0)),
        ],
        out_specs=pl.BlockSpec((BR, D_OUT), lambda i: (i, 0)),
        out_shape=jax.ShapeDtypeStruct((N, D_OUT), jnp.float32),
    )(pacc, ht, h0, dis, W3, b3)


def kernel(x, edge_index, W1, b1, W2, b2, W3, b3):
    row3 = edge_index[0].astype(jnp.int32).reshape(NW, NCHUNK, CHUNK)
    col3 = edge_index[1].astype(jnp.int32).reshape(NW, NCHUNK, CHUNK)

    h = _mlp(x, W1, b1.reshape(1, D_HID), W2, b2.reshape(1, D_HID))
    degp = _deg_sc(col3)
    dis, ht = _prep(degp, h)

    h0 = h
    for _ in range(KSTEPS - 1):
        pacc = _scatter_sc(row3, col3, ht)
        ht = _combine(pacc, ht, h0, dis)
    pacc = _scatter_sc(row3, col3, ht)
    return _final(pacc, ht, h0, dis, W3, b3.reshape(1, D_OUT))


# speedup vs baseline: 11.9358x; 11.9358x over previous
import functools

import jax
import jax.numpy as jnp
from jax import lax
from jax.experimental import pallas as pl
from jax.experimental.pallas import tpu as pltpu
from jax.experimental.pallas import tpu_sc as plsc

N = 10000
E = 320000
D_IN = 128
D_HID = 128
D_OUT = 64
KSTEPS = 10
ALPHA = 0.1

NC = 2
NS = 16
NW = NC * NS
EPW = E // NW
CHUNK = 80
NCHUNK = EPW // CHUNK
RPT = N // NS
ZROWS = 25

BR = 400
GRID = N // BR


def _mesh():
    return plsc.VectorSubcoreMesh(core_axis_name="c", subcore_axis_name="s")


def _deg_body(col3, degp, cbuf, dloc):
    c = lax.axis_index("c")
    s = lax.axis_index("s")
    wid = c * NS + s
    ones16 = jnp.full((16,), 1.0, jnp.float32)

    def zero_body(r, carry):
        dloc[pl.ds(r * 16, 16)] = jnp.zeros((16,), jnp.float32)
        return carry

    lax.fori_loop(0, N // 16, zero_body, 0)
    pltpu.sync_copy(col3.at[wid], cbuf)

    def body(i, carry):
        for j in range(CHUNK // 16):
            idx16 = cbuf[i, pl.ds(j * 16, 16)]
            plsc.addupdate_scatter(dloc, [idx16], ones16)
        return carry

    lax.fori_loop(0, NCHUNK, body, 0)
    for g in range(GRID):
        pltpu.sync_copy(dloc.at[pl.ds(g * BR, BR)], degp.at[g, wid])


def _deg_sc(col3):
    k = pl.kernel(
        _deg_body,
        out_type=jax.ShapeDtypeStruct((GRID, NW, BR), jnp.float32),
        mesh=_mesh(),
        scratch_types=[
            pltpu.VMEM((NCHUNK, CHUNK), jnp.int32),
            pltpu.VMEM((N,), jnp.float32),
        ],
        compiler_params=pltpu.CompilerParams(
            needs_layout_passes=False, use_tc_tiling_on_sc=False),
    )
    return k(col3)


def _scatter_body(row3, col3, ht_h, pacc, row_v, col_v, rows_v, zer_v,
                  acc_sh, sem):
    c = lax.axis_index("c")
    s = lax.axis_index("s")
    wid = c * NS + s

    for r in range(ZROWS):
        for j in range(D_HID // 16):
            zer_v[r, pl.ds(j * 16, 16)] = jnp.zeros((16,), jnp.float32)
    for j in range(RPT // ZROWS):
        pltpu.sync_copy(zer_v, acc_sh.at[pl.ds(s * RPT + j * ZROWS, ZROWS)])
    pltpu.sync_copy(row3.at[wid], row_v)
    pltpu.sync_copy(col3.at[wid], col_v)
    plsc.subcore_barrier()

    def body(i, carry):
        pltpu.async_copy(ht_h.at[row_v.at[i]], rows_v, sem).wait()
        pltpu.sync_copy(rows_v, acc_sh.at[col_v.at[i]], add=True)
        return carry

    lax.fori_loop(0, NCHUNK, body, 0)
    plsc.subcore_barrier()
    pltpu.sync_copy(acc_sh.at[pl.ds(s * RPT, RPT)],
                    pacc.at[c, pl.ds(s * RPT, RPT)])


def _scatter_sc(row3, col3, ht):
    k = pl.kernel(
        _scatter_body,
        out_type=jax.ShapeDtypeStruct((NC, N, D_HID), jnp.float32),
        mesh=_mesh(),
        scratch_types=[
            pltpu.VMEM((NCHUNK, CHUNK), jnp.int32),
            pltpu.VMEM((NCHUNK, CHUNK), jnp.int32),
            pltpu.VMEM((CHUNK, D_HID), jnp.float32),
            pltpu.VMEM((ZROWS, D_HID), jnp.float32),
            pltpu.VMEM_SHARED((N, D_HID), jnp.float32),
            pltpu.SemaphoreType.DMA,
        ],
        compiler_params=pltpu.CompilerParams(
            needs_layout_passes=False, use_tc_tiling_on_sc=False),
    )
    return k(row3, col3, ht)


def _mlp_body(x_ref, w1_ref, b1_ref, w2_ref, b2_ref, h_ref):
    h = jnp.maximum(
        jnp.dot(x_ref[...], w1_ref[...], preferred_element_type=jnp.float32)
        + b1_ref[...], 0.0)
    h_ref[...] = (
        jnp.dot(h, w2_ref[...], preferred_element_type=jnp.float32)
        + b2_ref[...])


def _mlp(x, W1, b1, W2, b2):
    return pl.pallas_call(
        _mlp_body,
        grid=(GRID,),
        in_specs=[
            pl.BlockSpec((BR, D_IN), lambda i: (i, 0)),
            pl.BlockSpec((D_IN, D_HID), lambda i: (0, 0)),
            pl.BlockSpec((1, D_HID), lambda i: (0, 0)),
            pl.BlockSpec((D_HID, D_HID), lambda i: (0, 0)),
            pl.BlockSpec((1, D_HID), lambda i: (0, 0)),
        ],
        out_specs=pl.BlockSpec((BR, D_HID), lambda i: (i, 0)),
        out_shape=jax.ShapeDtypeStruct((N, D_HID), jnp.float32),
    )(x, W1, b1, W2, b2)


def _prep_body(degp_ref, h_ref, dis_ref, ht_ref):
    ones = jnp.ones((NW, 1), jnp.float32)
    deg = lax.dot_general(degp_ref[0], ones, (((0,), (0,)), ((), ())),
                          preferred_element_type=jnp.float32)
    dis = lax.rsqrt(deg + 1.0)
    dis_ref[...] = dis
    ht_ref[...] = dis * h_ref[...]


def _prep(degp, h):
    return pl.pallas_call(
        _prep_body,
        grid=(GRID,),
        in_specs=[
            pl.BlockSpec((1, NW, BR), lambda i: (i, 0, 0)),
            pl.BlockSpec((BR, D_HID), lambda i: (i, 0)),
        ],
        out_specs=[
            pl.BlockSpec((BR, 1), lambda i: (i, 0)),
            pl.BlockSpec((BR, D_HID), lambda i: (i, 0)),
        ],
        out_shape=[
            jax.ShapeDtypeStruct((N, 1), jnp.float32),
            jax.ShapeDtypeStruct((N, D_HID), jnp.float32),
        ],
    )(degp, h)


def _combine_body(pacc_ref, ht_ref, h0_ref, dis_ref, out_ref):
    acc = pacc_ref[0] + pacc_ref[1]
    hk = ((1.0 - ALPHA) * dis_ref[...] * (acc + ht_ref[...])
          + ALPHA * h0_ref[...])
    out_ref[...] = dis_ref[...] * hk


def _combine(pacc, ht, h0, dis):
    return pl.pallas_call(
        _combine_body,
        grid=(GRID,),
        in_specs=[
            pl.BlockSpec((NC, BR, D_HID), lambda i: (0, i, 0)),
            pl.BlockSpec((BR, D_HID), lambda i: (i, 0)),
            pl.BlockSpec((BR, D_HID), lambda i: (i, 0)),
            pl.BlockSpec((BR, 1), lambda i: (i, 0)),
        ],
        out_specs=pl.BlockSpec((BR, D_HID), lambda i: (i, 0)),
        out_shape=jax.ShapeDtypeStruct((N, D_HID), jnp.float32),
    )(pacc, ht, h0, dis)


def _final_body(pacc_ref, ht_ref, h0_ref, dis_ref, w3_ref, b3_ref, out_ref):
    acc = pacc_ref[0] + pacc_ref[1]
    hk = ((1.0 - ALPHA) * dis_ref[...] * (acc + ht_ref[...])
          + ALPHA * h0_ref[...])
    out_ref[...] = (
        jnp.dot(hk, w3_ref[...], preferred_element_type=jnp.float32)
        + b3_ref[...])


def _final(pacc, ht, h0, dis, W3, b3):
    return pl.pallas_call(
        _final_body,
        grid=(GRID,),
        in_specs=[
            pl.BlockSpec((NC, BR, D_HID), lambda i: (0, i, 0)),
            pl.BlockSpec((BR, D_HID), lambda i: (i, 0)),
            pl.BlockSpec((BR, D_HID), lambda i: (i, 0)),
            pl.BlockSpec((BR, 1), lambda i: (i, 0)),
            pl.BlockSpec((D_HID, D_OUT), lambda i: (0, 0)),
            pl.BlockSpec((1, D_OUT), lambda i: (0, 0)),
        ],
        out_specs=pl.BlockSpec((BR, D_OUT), lambda i: (i, 0)),
        out_shape=jax.ShapeDtypeStruct((N, D_OUT), jnp.float32),
    )(pacc, ht, h0, dis, W3, b3)


def kernel(x, edge_index, W1, b1, W2, b2, W3, b3):
    row3 = edge_index[0].astype(jnp.int32).reshape(NW, NCHUNK, CHUNK)
    col3 = edge_index[1].astype(jnp.int32).reshape(NW, NCHUNK, CHUNK)

    h = _mlp(x, W1, b1.reshape(1, D_HID), W2, b2.reshape(1, D_HID))
    degp = _deg_sc(col3)
    dis, ht = _prep(degp, h)

    h0 = h
    for _ in range(KSTEPS - 1):
        pacc = _scatter_sc(row3, col3, ht)
        ht = _combine(pacc, ht, h0, dis)
    pacc = _scatter_sc(row3, col3, ht)
    return _final(pacc, ht, h0, dis, W3, b3.reshape(1, D_OUT))


# speedup vs baseline: 17.7524x; 1.4873x over previous
import functools

import jax
import jax.numpy as jnp
from jax import lax
from jax.experimental import pallas as pl
from jax.experimental.pallas import tpu as pltpu
from jax.experimental.pallas import tpu_sc as plsc

N = 10000
E = 320000
D_IN = 128
D_HID = 128
D_OUT = 64
KSTEPS = 10
ALPHA = 0.1

NC = 2
NS = 16
NW = NC * NS
EPW = E // NW
CHUNK = 80
NCHUNK = EPW // CHUNK
RPT = N // NS
ZROWS = 25

BR = 400
GRID = N // BR


def _mesh():
    return plsc.VectorSubcoreMesh(core_axis_name="c", subcore_axis_name="s")


def _deg_body(col3, degp, cbuf, dloc):
    c = lax.axis_index("c")
    s = lax.axis_index("s")
    wid = c * NS + s
    ones16 = jnp.full((16,), 1.0, jnp.float32)

    def zero_body(r, carry):
        dloc[pl.ds(r * 16, 16)] = jnp.zeros((16,), jnp.float32)
        return carry

    lax.fori_loop(0, N // 16, zero_body, 0)
    pltpu.sync_copy(col3.at[wid], cbuf)

    def body(i, carry):
        for j in range(CHUNK // 16):
            idx16 = cbuf[i, pl.ds(j * 16, 16)]
            plsc.addupdate_scatter(dloc, [idx16], ones16)
        return carry

    lax.fori_loop(0, NCHUNK, body, 0)
    for g in range(GRID):
        pltpu.sync_copy(dloc.at[pl.ds(g * BR, BR)], degp.at[g, wid])


def _deg_sc(col3):
    k = pl.kernel(
        _deg_body,
        out_type=jax.ShapeDtypeStruct((GRID, NW, BR), jnp.float32),
        mesh=_mesh(),
        scratch_types=[
            pltpu.VMEM((NCHUNK, CHUNK), jnp.int32),
            pltpu.VMEM((N,), jnp.float32),
        ],
        compiler_params=pltpu.CompilerParams(
            needs_layout_passes=False, use_tc_tiling_on_sc=False),
    )
    return k(col3)


NBUF = 5
OUTER = NCHUNK // NBUF
DH = D_HID // 2


def _scatter_body(row3, col3, ht_h, pacc, row_v, col_v, rows_v, zer_v,
                  acc_sh, gsem, ssem):
    c = lax.axis_index("c")
    s = lax.axis_index("s")
    wid = c * NS + s

    for r in range(ZROWS):
        for j in range(DH // 16):
            zer_v[r, pl.ds(j * 16, 16)] = jnp.zeros((16,), jnp.float32)
    pltpu.sync_copy(row3.at[wid], row_v)
    pltpu.sync_copy(col3.at[wid], col_v)

    for f in range(D_HID // DH):
        for j in range(RPT // ZROWS):
            pltpu.sync_copy(zer_v,
                            acc_sh.at[pl.ds(s * RPT + j * ZROWS, ZROWS)])
        plsc.subcore_barrier()

        def gstart(b, i):
            pltpu.async_copy(ht_h.at[f].at[row_v.at[i]],
                             rows_v.at[b], gsem.at[b])

        def gwait(b):
            pltpu.make_async_copy(ht_h.at[f].at[row_v.at[0]],
                                  rows_v.at[b], gsem.at[b]).wait()

        def sstart(b, i):
            pltpu.async_copy(rows_v.at[b], acc_sh.at[col_v.at[i]],
                             ssem.at[b], add=True)

        def swait(b):
            pltpu.make_async_copy(rows_v.at[b], acc_sh.at[col_v.at[0]],
                                  ssem.at[b]).wait()

        for b in range(NBUF):
            gstart(b, b)

        def outer(g, carry):
            base = g * NBUF
            for b in range(NBUF):
                gwait(b)
                sstart(b, base + b)
            for b in range(NBUF):
                swait(b)
                gstart(b, base + b + NBUF)
            return carry

        lax.fori_loop(0, OUTER - 1, outer, 0)
        base = (OUTER - 1) * NBUF
        for b in range(NBUF):
            gwait(b)
            sstart(b, base + b)
        for b in range(NBUF):
            swait(b)
        plsc.subcore_barrier()
        pltpu.sync_copy(acc_sh.at[pl.ds(s * RPT, RPT)],
                        pacc.at[c, pl.ds(s * RPT, RPT), pl.ds(f * DH, DH)])


def _scatter_sc(row3, col3, ht2):
    k = pl.kernel(
        _scatter_body,
        out_type=jax.ShapeDtypeStruct((NC, N, D_HID), jnp.float32),
        mesh=_mesh(),
        scratch_types=[
            pltpu.VMEM((NCHUNK, CHUNK), jnp.int32),
            pltpu.VMEM((NCHUNK, CHUNK), jnp.int32),
            pltpu.VMEM((NBUF, CHUNK, DH), jnp.float32),
            pltpu.VMEM((ZROWS, DH), jnp.float32),
            pltpu.VMEM_SHARED((N, DH), jnp.float32),
            pltpu.SemaphoreType.DMA((NBUF,)),
            pltpu.SemaphoreType.DMA((NBUF,)),
        ],
        compiler_params=pltpu.CompilerParams(
            needs_layout_passes=False, use_tc_tiling_on_sc=False),
    )
    return k(row3, col3, ht2)


def _mlp_body(x_ref, w1_ref, b1_ref, w2_ref, b2_ref, h_ref):
    h = jnp.maximum(
        jnp.dot(x_ref[...], w1_ref[...], preferred_element_type=jnp.float32)
        + b1_ref[...], 0.0)
    h_ref[...] = (
        jnp.dot(h, w2_ref[...], preferred_element_type=jnp.float32)
        + b2_ref[...])


def _mlp(x, W1, b1, W2, b2):
    return pl.pallas_call(
        _mlp_body,
        grid=(GRID,),
        in_specs=[
            pl.BlockSpec((BR, D_IN), lambda i: (i, 0)),
            pl.BlockSpec((D_IN, D_HID), lambda i: (0, 0)),
            pl.BlockSpec((1, D_HID), lambda i: (0, 0)),
            pl.BlockSpec((D_HID, D_HID), lambda i: (0, 0)),
            pl.BlockSpec((1, D_HID), lambda i: (0, 0)),
        ],
        out_specs=pl.BlockSpec((BR, D_HID), lambda i: (i, 0)),
        out_shape=jax.ShapeDtypeStruct((N, D_HID), jnp.float32),
    )(x, W1, b1, W2, b2)


def _prep_body(degp_ref, h_ref, dis_ref, ht_ref):
    ones = jnp.ones((NW, 1), jnp.float32)
    deg = lax.dot_general(degp_ref[0], ones, (((0,), (0,)), ((), ())),
                          preferred_element_type=jnp.float32)
    dis = lax.rsqrt(deg + 1.0)
    dis_ref[...] = dis
    ht = dis * h_ref[...]
    ht_ref[0] = ht[:, :DH]
    ht_ref[1] = ht[:, DH:]


def _prep(degp, h):
    return pl.pallas_call(
        _prep_body,
        grid=(GRID,),
        in_specs=[
            pl.BlockSpec((1, NW, BR), lambda i: (i, 0, 0)),
            pl.BlockSpec((BR, D_HID), lambda i: (i, 0)),
        ],
        out_specs=[
            pl.BlockSpec((BR, 1), lambda i: (i, 0)),
            pl.BlockSpec((2, BR, DH), lambda i: (0, i, 0)),
        ],
        out_shape=[
            jax.ShapeDtypeStruct((N, 1), jnp.float32),
            jax.ShapeDtypeStruct((2, N, DH), jnp.float32),
        ],
    )(degp, h)


def _combine_body(pacc_ref, ht_ref, h0_ref, dis_ref, out_ref):
    acc = pacc_ref[0] + pacc_ref[1]
    htf = jnp.concatenate([ht_ref[0], ht_ref[1]], axis=-1)
    hk = ((1.0 - ALPHA) * dis_ref[...] * (acc + htf)
          + ALPHA * h0_ref[...])
    htn = dis_ref[...] * hk
    out_ref[0] = htn[:, :DH]
    out_ref[1] = htn[:, DH:]


def _combine(pacc, ht2, h0, dis):
    return pl.pallas_call(
        _combine_body,
        grid=(GRID,),
        in_specs=[
            pl.BlockSpec((NC, BR, D_HID), lambda i: (0, i, 0)),
            pl.BlockSpec((2, BR, DH), lambda i: (0, i, 0)),
            pl.BlockSpec((BR, D_HID), lambda i: (i, 0)),
            pl.BlockSpec((BR, 1), lambda i: (i, 0)),
        ],
        out_specs=pl.BlockSpec((2, BR, DH), lambda i: (0, i, 0)),
        out_shape=jax.ShapeDtypeStruct((2, N, DH), jnp.float32),
    )(pacc, ht2, h0, dis)


def _final_body(pacc_ref, ht_ref, h0_ref, dis_ref, w3_ref, b3_ref, out_ref):
    acc = pacc_ref[0] + pacc_ref[1]
    htf = jnp.concatenate([ht_ref[0], ht_ref[1]], axis=-1)
    hk = ((1.0 - ALPHA) * dis_ref[...] * (acc + htf)
          + ALPHA * h0_ref[...])
    out_ref[...] = (
        jnp.dot(hk, w3_ref[...], preferred_element_type=jnp.float32)
        + b3_ref[...])


def _final(pacc, ht2, h0, dis, W3, b3):
    return pl.pallas_call(
        _final_body,
        grid=(GRID,),
        in_specs=[
            pl.BlockSpec((NC, BR, D_HID), lambda i: (0, i, 0)),
            pl.BlockSpec((2, BR, DH), lambda i: (0, i, 0)),
            pl.BlockSpec((BR, D_HID), lambda i: (i, 0)),
            pl.BlockSpec((BR, 1), lambda i: (i, 0)),
            pl.BlockSpec((D_HID, D_OUT), lambda i: (0, 0)),
            pl.BlockSpec((1, D_OUT), lambda i: (0, 0)),
        ],
        out_specs=pl.BlockSpec((BR, D_OUT), lambda i: (i, 0)),
        out_shape=jax.ShapeDtypeStruct((N, D_OUT), jnp.float32),
    )(pacc, ht2, h0, dis, W3, b3)


def kernel(x, edge_index, W1, b1, W2, b2, W3, b3):
    row3 = edge_index[0].astype(jnp.int32).reshape(NW, NCHUNK, CHUNK)
    col3 = edge_index[1].astype(jnp.int32).reshape(NW, NCHUNK, CHUNK)

    h = _mlp(x, W1, b1.reshape(1, D_HID), W2, b2.reshape(1, D_HID))
    degp = _deg_sc(col3)
    dis, ht = _prep(degp, h)

    h0 = h
    for _ in range(KSTEPS - 1):
        pacc = _scatter_sc(row3, col3, ht)
        ht = _combine(pacc, ht, h0, dis)
    pacc = _scatter_sc(row3, col3, ht)
    return _final(pacc, ht, h0, dis, W3, b3.reshape(1, D_OUT))
